# baseline (speedup 1.0000x reference)
import jax
import jax.numpy as jnp
from jax import lax
from jax.experimental import pallas as pl
from jax.experimental.pallas import tpu as pltpu
from jax.experimental.pallas import tpu_sc as plsc

BATCH = 4096
SEQ_LEN = 2048
ALPHABET = 5
LANES = 16

NUM_CORES = 2
NUM_SUBCORES = 16
NUM_WORKERS = NUM_CORES * NUM_SUBCORES
ROWS_PER_WORKER = BATCH // NUM_WORKERS
OUT_ROW = SEQ_LEN * ALPHABET

RGRP = 4
NG = ROWS_PER_WORKER // RGRP
NP = NG // 2
SEQ_G = RGRP * SEQ_LEN
OUT_G = RGRP * OUT_ROW
BLOCKS_G = SEQ_G // LANES


def _sc_body(seq_hbm, tbl_hbm, out_hbm,
             seq0, seq1, out0, out1, tbl_v, si0, si1, so0, so1):
    wid = lax.axis_index("s") * NUM_CORES + lax.axis_index("c")
    seq_base = wid * (ROWS_PER_WORKER * SEQ_LEN)
    out_base = wid * (ROWS_PER_WORKER * OUT_ROW)
    seqb, outb = (seq0, seq1), (out0, out1)
    sis, sos = (si0, si1), (so0, so1)

    pltpu.sync_copy(tbl_hbm, tbl_v)

    def load(g, s):
        pltpu.async_copy(
            seq_hbm.at[pl.ds(seq_base + g * SEQ_G, SEQ_G)], seqb[s], sis[s])

    def wait_load(g, s):
        pltpu.make_async_copy(
            seq_hbm.at[pl.ds(seq_base + g * SEQ_G, SEQ_G)], seqb[s],
            sis[s]).wait()

    def store(g, s):
        pltpu.async_copy(
            outb[s], out_hbm.at[pl.ds(out_base + g * OUT_G, OUT_G)], sos[s])

    def wait_store(g, s):
        pltpu.make_async_copy(
            outb[s], out_hbm.at[pl.ds(out_base + g * OUT_G, OUT_G)],
            sos[s]).wait()

    def compute(sv, ov):
        def blk_body(b, _):
            sbase = b * LANES
            obase = b * (LANES * ALPHABET)
            for v in range(ALPHABET):
                j = lax.iota(jnp.int32, LANES) + (LANES * v)
                pat_l = lax.shift_right_logical(j * 52429, 18)
                pat_k = j - pat_l * ALPHABET
                sg = plsc.load_gather(sv, [sbase + pat_l])
                val = plsc.load_gather(tbl_v, [sg * ALPHABET + pat_k])
                ov[pl.ds(obase + v * LANES, LANES)] = val
            return ()

        lax.fori_loop(0, BLOCKS_G, blk_body, (), unroll=4)

    load(0, 0)
    load(1, 1)

    def pair_body(p, _):
        for s in range(2):
            g = 2 * p + s
            wait_load(g, s)

            @pl.when(g >= 2)
            def _():
                wait_store(g - 2, s)

            compute(seqb[s], outb[s])
            store(g, s)

            @pl.when(g + 2 < NG)
            def _():
                load(g + 2, s)
        return ()

    lax.fori_loop(0, NP, pair_body, ())
    wait_store(NG - 2, 0)
    wait_store(NG - 1, 1)


@jax.jit
def _encode(seq, tbl_pad):
    mesh = plsc.VectorSubcoreMesh(core_axis_name="c", subcore_axis_name="s")
    run = pl.kernel(
        _sc_body,
        out_type=jax.ShapeDtypeStruct((BATCH * OUT_ROW,), jnp.float32),
        mesh=mesh,
        compiler_params=pltpu.CompilerParams(needs_layout_passes=False),
        scratch_types=[
            pltpu.VMEM((SEQ_G,), jnp.int32),
            pltpu.VMEM((SEQ_G,), jnp.int32),
            pltpu.VMEM((OUT_G,), jnp.float32),
            pltpu.VMEM((OUT_G,), jnp.float32),
            pltpu.VMEM((32,), jnp.float32),
            pltpu.SemaphoreType.DMA,
            pltpu.SemaphoreType.DMA,
            pltpu.SemaphoreType.DMA,
            pltpu.SemaphoreType.DMA,
        ],
    )
    return run(seq, tbl_pad)


def kernel(sequences, onehot_matrix):
    seq = sequences.astype(jnp.int32).reshape(-1)
    tbl_pad = jnp.pad(onehot_matrix.reshape(-1).astype(jnp.float32), (0, 7))
    out = _encode(seq, tbl_pad)
    return out.reshape(BATCH, SEQ_LEN, ALPHABET)


# speedup vs baseline: 4.6125x; 4.6125x over previous
import jax
import jax.numpy as jnp
from jax import lax
from jax.experimental import pallas as pl
from jax.experimental.pallas import tpu as pltpu
from jax.experimental.pallas import tpu_sc as plsc

BATCH = 4096
SEQ_LEN = 2048
ALPHABET = 5
LANES = 16

NUM_CORES = 2
NUM_SUBCORES = 16
NUM_WORKERS = NUM_CORES * NUM_SUBCORES
ROWS_PER_WORKER = BATCH // NUM_WORKERS
OUT_ROW = SEQ_LEN * ALPHABET
NUM_BLOCKS = SEQ_LEN // LANES


def _sc_body(seq_hbm, tbl_hbm, out_hbm,
             seq0, seq1, out0, out1, tbl_v, si0, si1, so0, so1):
    wid = lax.axis_index("s") * NUM_CORES + lax.axis_index("c")
    row0 = wid * ROWS_PER_WORKER
    seqb, outb = (seq0, seq1), (out0, out1)
    sis, sos = (si0, si1), (so0, so1)

    pltpu.sync_copy(tbl_hbm, tbl_v)

    def load(r, s):
        pltpu.async_copy(seq_hbm.at[row0 + r], seqb[s], sis[s])

    def wait_load(r, s):
        pltpu.make_async_copy(seq_hbm.at[row0 + r], seqb[s], sis[s]).wait()

    def store(r, s):
        pltpu.async_copy(outb[s], out_hbm.at[row0 + r], sos[s])

    def wait_store(r, s):
        pltpu.make_async_copy(outb[s], out_hbm.at[row0 + r], sos[s]).wait()

    def compute(sv, ov):
        def blk_body(b, _):
            sbase = b * LANES
            obase = b * (LANES * ALPHABET)
            for v in range(ALPHABET):
                j = lax.iota(jnp.int32, LANES) + (LANES * v)
                pat_l = lax.shift_right_logical(j * 52429, 18)
                pat_k = j - pat_l * ALPHABET
                sg = plsc.load_gather(sv, [sbase + pat_l])
                val = plsc.load_gather(tbl_v, [sg * ALPHABET + pat_k])
                ov[pl.ds(obase + v * LANES, LANES)] = val
            return ()

        lax.fori_loop(0, NUM_BLOCKS, blk_body, (), unroll=4)

    load(0, 0)
    load(1, 1)

    def pair_body(p, _):
        for s in range(2):
            r = 2 * p + s
            wait_load(r, s)

            @pl.when(r >= 2)
            def _():
                wait_store(r - 2, s)

            compute(seqb[s], outb[s])
            store(r, s)

            @pl.when(r + 2 < ROWS_PER_WORKER)
            def _():
                load(r + 2, s)
        return ()

    lax.fori_loop(0, ROWS_PER_WORKER // 2, pair_body, ())
    wait_store(ROWS_PER_WORKER - 2, 0)
    wait_store(ROWS_PER_WORKER - 1, 1)


@jax.jit
def _encode(seq, tbl_pad):
    mesh = plsc.VectorSubcoreMesh(core_axis_name="c", subcore_axis_name="s")
    run = pl.kernel(
        _sc_body,
        out_type=jax.ShapeDtypeStruct((BATCH, OUT_ROW), jnp.float32),
        mesh=mesh,
        compiler_params=pltpu.CompilerParams(needs_layout_passes=False),
        scratch_types=[
            pltpu.VMEM((SEQ_LEN,), jnp.int32),
            pltpu.VMEM((SEQ_LEN,), jnp.int32),
            pltpu.VMEM((OUT_ROW,), jnp.float32),
            pltpu.VMEM((OUT_ROW,), jnp.float32),
            pltpu.VMEM((32,), jnp.float32),
            pltpu.SemaphoreType.DMA,
            pltpu.SemaphoreType.DMA,
            pltpu.SemaphoreType.DMA,
            pltpu.SemaphoreType.DMA,
        ],
    )
    return run(seq, tbl_pad)


def kernel(sequences, onehot_matrix):
    seq = sequences.astype(jnp.int32)
    tbl_pad = jnp.pad(onehot_matrix.reshape(-1).astype(jnp.float32), (0, 7))
    out = _encode(seq, tbl_pad)
    return out.reshape(BATCH, SEQ_LEN, ALPHABET)


# speedup vs baseline: 15.5519x; 3.3717x over previous
import jax
import jax.numpy as jnp
from jax import lax
from jax.experimental import pallas as pl
from jax.experimental.pallas import tpu as pltpu
from jax.experimental.pallas import tpu_sc as plsc

BATCH = 4096
SEQ_LEN = 2048
ALPHABET = 5
LANES = 16

NUM_CORES = 2
NUM_SUBCORES = 16
NUM_WORKERS = NUM_CORES * NUM_SUBCORES

TILE_R = 8
TILE_C = 1024
COLS_TILES = SEQ_LEN // TILE_C
NG = (BATCH // TILE_R) * COLS_TILES // NUM_WORKERS
BLOCKS_G = TILE_R * TILE_C // LANES
CBLK = TILE_C // LANES


def _sc_body(seq_hbm, tbl_hbm, out_hbm,
             seq0, seq1, out0, out1, tbl_v, si0, si1, so0, so1):
    wid = lax.axis_index("s") * NUM_CORES + lax.axis_index("c")
    g0 = wid * NG
    seqb, outb = (seq0, seq1), (out0, out1)
    sis, sos = (si0, si1), (so0, so1)

    pltpu.sync_copy(tbl_hbm, tbl_v)

    def tile_origin(g):
        gg = g0 + g
        r0 = (gg // COLS_TILES) * TILE_R
        c0 = (gg % COLS_TILES) * TILE_C
        return r0, c0

    def load(g, s):
        r0, c0 = tile_origin(g)
        pltpu.async_copy(
            seq_hbm.at[pl.ds(r0, TILE_R), pl.ds(c0, TILE_C)], seqb[s], sis[s])

    def wait_load(g, s):
        r0, c0 = tile_origin(g)
        pltpu.make_async_copy(
            seq_hbm.at[pl.ds(r0, TILE_R), pl.ds(c0, TILE_C)], seqb[s],
            sis[s]).wait()

    def store(g, s):
        r0, c0 = tile_origin(g)
        pltpu.async_copy(
            outb[s], out_hbm.at[:, pl.ds(r0, TILE_R), pl.ds(c0, TILE_C)],
            sos[s])

    def wait_store(g, s):
        r0, c0 = tile_origin(g)
        pltpu.make_async_copy(
            outb[s], out_hbm.at[:, pl.ds(r0, TILE_R), pl.ds(c0, TILE_C)],
            sos[s]).wait()

    def compute(sv, ov):
        def blk_body(b, _):
            r = b >> 6
            c = (b & (CBLK - 1)) << 4
            sg = sv[r, pl.ds(c, LANES)]
            for k in range(ALPHABET):
                val = plsc.load_gather(tbl_v, [sg + (8 * k)])
                ov[k, r, pl.ds(c, LANES)] = val
            return ()

        lax.fori_loop(0, BLOCKS_G, blk_body, (), unroll=4)

    load(0, 0)
    load(1, 1)

    def pair_body(p, _):
        for s in range(2):
            g = 2 * p + s
            wait_load(g, s)

            @pl.when(g >= 2)
            def _():
                wait_store(g - 2, s)

            compute(seqb[s], outb[s])
            store(g, s)

            @pl.when(g + 2 < NG)
            def _():
                load(g + 2, s)
        return ()

    lax.fori_loop(0, NG // 2, pair_body, ())
    wait_store(NG - 2, 0)
    wait_store(NG - 1, 1)


@jax.jit
def _encode(seq, tblT_pad):
    mesh = plsc.VectorSubcoreMesh(core_axis_name="c", subcore_axis_name="s")
    run = pl.kernel(
        _sc_body,
        out_type=jax.ShapeDtypeStruct((ALPHABET, BATCH, SEQ_LEN), jnp.float32),
        mesh=mesh,
        compiler_params=pltpu.CompilerParams(needs_layout_passes=False),
        scratch_types=[
            pltpu.VMEM((TILE_R, TILE_C), jnp.int32),
            pltpu.VMEM((TILE_R, TILE_C), jnp.int32),
            pltpu.VMEM((ALPHABET, TILE_R, TILE_C), jnp.float32),
            pltpu.VMEM((ALPHABET, TILE_R, TILE_C), jnp.float32),
            pltpu.VMEM((64,), jnp.float32),
            pltpu.SemaphoreType.DMA,
            pltpu.SemaphoreType.DMA,
            pltpu.SemaphoreType.DMA,
            pltpu.SemaphoreType.DMA,
        ],
    )
    return run(seq, tblT_pad)


def kernel(sequences, onehot_matrix):
    seq = sequences.astype(jnp.int32)
    tblT = jnp.pad(onehot_matrix.T.astype(jnp.float32), ((0, 0), (0, 3)))
    tblT_pad = jnp.pad(tblT.reshape(-1), (0, 24))
    out5 = _encode(seq, tblT_pad)
    return jnp.transpose(out5, (1, 2, 0))
